# baseline (speedup 1.0000x reference)
import functools

import jax
import jax.numpy as jnp
from jax.experimental import pallas as pl
from jax.experimental.pallas import tpu as pltpu

BN = 512
N_CHUNKS = 16


def _dot_bias(x_chunk, w, b):
    acc = jax.lax.dot_general(
        x_chunk,
        w,
        dimension_numbers=(((1,), (1,)), ((), ())),
        preferred_element_type=jnp.float32,
    )
    return acc + b


def _matmul_bias_kernel(x_hbm, w_ref, b_ref, o_ref, x_bf16, stage, sems):
    M = x_bf16.shape[0]
    chunk = M // N_CHUNKS

    @pl.when(pl.program_id(0) == 0)
    def _load_x_and_compute():
        w = w_ref[...].astype(jnp.bfloat16)
        copies = []
        for c in range(N_CHUNKS):
            rows = pl.ds(c * chunk, chunk)
            copies.append(
                pltpu.make_async_copy(x_hbm.at[rows, :], stage.at[c % 2], sems.at[c])
            )
        copies[0].start()
        copies[1].start()
        for c in range(N_CHUNKS):
            rows = pl.ds(c * chunk, chunk)
            copies[c].wait()
            x_bf16[rows, :] = stage[c % 2].astype(jnp.bfloat16)
            if c + 2 < N_CHUNKS:
                copies[c + 2].start()
            o_ref[rows, :] = _dot_bias(x_bf16[rows, :], w, b_ref[...])

    @pl.when(pl.program_id(0) > 0)
    def _compute():
        o_ref[...] = _dot_bias(
            x_bf16[...], w_ref[...].astype(jnp.bfloat16), b_ref[...]
        )


@functools.partial(jax.jit, static_argnums=())
def kernel(x, W, b):
    M, K = x.shape
    N = W.shape[0]
    b2 = b.reshape(1, N)
    return pl.pallas_call(
        _matmul_bias_kernel,
        grid=(N // BN,),
        in_specs=[
            pl.BlockSpec(memory_space=pl.ANY),
            pl.BlockSpec((BN, K), lambda j: (j, 0)),
            pl.BlockSpec((1, BN), lambda j: (0, j)),
        ],
        out_specs=pl.BlockSpec((M, BN), lambda j: (0, j)),
        out_shape=jax.ShapeDtypeStruct((M, N), jnp.float32),
        scratch_shapes=[
            pltpu.VMEM((M, K), jnp.bfloat16),
            pltpu.VMEM((2, M // N_CHUNKS, K), jnp.float32),
            pltpu.SemaphoreType.DMA((N_CHUNKS,)),
        ],
        compiler_params=pltpu.CompilerParams(
            dimension_semantics=("arbitrary",),
            vmem_limit_bytes=128 * 1024 * 1024,
        ),
    )(x, W, b2)


# speedup vs baseline: 1.0221x; 1.0221x over previous
import functools

import jax
import jax.numpy as jnp
from jax.experimental import pallas as pl
from jax.experimental.pallas import tpu as pltpu

BN = 512
N_CHUNKS = 8


def _dot_bias(x_chunk, w, b):
    acc = jax.lax.dot_general(
        x_chunk,
        w,
        dimension_numbers=(((1,), (1,)), ((), ())),
        preferred_element_type=jnp.float32,
    )
    return acc + b


def _matmul_bias_kernel(x_hbm, w_ref, b_ref, o_ref, x_bf16, stage, sems):
    M = x_bf16.shape[0]
    chunk = M // N_CHUNKS

    @pl.when(pl.program_id(0) == 0)
    def _load_x_and_compute():
        w = w_ref[...].astype(jnp.bfloat16)
        copies = []
        for c in range(N_CHUNKS):
            rows = pl.ds(c * chunk, chunk)
            copies.append(
                pltpu.make_async_copy(x_hbm.at[rows, :], stage.at[c % 2], sems.at[c])
            )
        copies[0].start()
        copies[1].start()
        for c in range(N_CHUNKS):
            rows = pl.ds(c * chunk, chunk)
            copies[c].wait()
            x_bf16[rows, :] = stage[c % 2].astype(jnp.bfloat16)
            if c + 2 < N_CHUNKS:
                copies[c + 2].start()
            o_ref[rows, :] = _dot_bias(x_bf16[rows, :], w, b_ref[...])

    @pl.when(pl.program_id(0) > 0)
    def _compute():
        o_ref[...] = _dot_bias(
            x_bf16[...], w_ref[...].astype(jnp.bfloat16), b_ref[...]
        )


@functools.partial(jax.jit, static_argnums=())
def kernel(x, W, b):
    M, K = x.shape
    N = W.shape[0]
    b2 = b.reshape(1, N)
    return pl.pallas_call(
        _matmul_bias_kernel,
        grid=(N // BN,),
        in_specs=[
            pl.BlockSpec(memory_space=pl.ANY),
            pl.BlockSpec((BN, K), lambda j: (j, 0)),
            pl.BlockSpec((1, BN), lambda j: (0, j)),
        ],
        out_specs=pl.BlockSpec((M, BN), lambda j: (0, j)),
        out_shape=jax.ShapeDtypeStruct((M, N), jnp.float32),
        scratch_shapes=[
            pltpu.VMEM((M, K), jnp.bfloat16),
            pltpu.VMEM((2, M // N_CHUNKS, K), jnp.float32),
            pltpu.SemaphoreType.DMA((N_CHUNKS,)),
        ],
        compiler_params=pltpu.CompilerParams(
            dimension_semantics=("arbitrary",),
            vmem_limit_bytes=128 * 1024 * 1024,
        ),
    )(x, W, b2)


# speedup vs baseline: 1.0293x; 1.0070x over previous
import functools

import jax
import jax.numpy as jnp
from jax.experimental import pallas as pl
from jax.experimental.pallas import tpu as pltpu

BN = 512
N_CHUNKS = 4


def _dot_bias(x_chunk, w, b):
    acc = jax.lax.dot_general(
        x_chunk,
        w,
        dimension_numbers=(((1,), (1,)), ((), ())),
        preferred_element_type=jnp.float32,
    )
    return acc + b


def _matmul_bias_kernel(x_hbm, w_ref, b_ref, o_ref, x_bf16, stage, sems):
    M = x_bf16.shape[0]
    chunk = M // N_CHUNKS

    @pl.when(pl.program_id(0) == 0)
    def _load_x_and_compute():
        w = w_ref[...].astype(jnp.bfloat16)
        copies = []
        for c in range(N_CHUNKS):
            rows = pl.ds(c * chunk, chunk)
            copies.append(
                pltpu.make_async_copy(x_hbm.at[rows, :], stage.at[c % 2], sems.at[c])
            )
        copies[0].start()
        copies[1].start()
        for c in range(N_CHUNKS):
            rows = pl.ds(c * chunk, chunk)
            copies[c].wait()
            x_bf16[rows, :] = stage[c % 2].astype(jnp.bfloat16)
            if c + 2 < N_CHUNKS:
                copies[c + 2].start()
            o_ref[rows, :] = _dot_bias(x_bf16[rows, :], w, b_ref[...])

    @pl.when(pl.program_id(0) > 0)
    def _compute():
        o_ref[...] = _dot_bias(
            x_bf16[...], w_ref[...].astype(jnp.bfloat16), b_ref[...]
        )


@functools.partial(jax.jit, static_argnums=())
def kernel(x, W, b):
    M, K = x.shape
    N = W.shape[0]
    b2 = b.reshape(1, N)
    return pl.pallas_call(
        _matmul_bias_kernel,
        grid=(N // BN,),
        in_specs=[
            pl.BlockSpec(memory_space=pl.ANY),
            pl.BlockSpec((BN, K), lambda j: (j, 0)),
            pl.BlockSpec((1, BN), lambda j: (0, j)),
        ],
        out_specs=pl.BlockSpec((M, BN), lambda j: (0, j)),
        out_shape=jax.ShapeDtypeStruct((M, N), jnp.float32),
        scratch_shapes=[
            pltpu.VMEM((M, K), jnp.bfloat16),
            pltpu.VMEM((2, M // N_CHUNKS, K), jnp.float32),
            pltpu.SemaphoreType.DMA((N_CHUNKS,)),
        ],
        compiler_params=pltpu.CompilerParams(
            dimension_semantics=("arbitrary",),
            vmem_limit_bytes=128 * 1024 * 1024,
        ),
    )(x, W, b2)
